# baseline (speedup 1.0000x reference)
import functools
import math

import jax
import jax.numpy as jnp
from jax import lax
from jax.experimental import pallas as pl
from jax.experimental.pallas import tpu as pltpu
from jax.experimental.pallas import tpu_sc as plsc

H_MIN, H_MAX = 2.0e5, 3.5e6
P_MIN, P_MAX = 1.0e3, 1.0e8
N_H, N_P = 1024, 1024
LOGP_MIN = math.log(P_MIN)
LOGP_MAX = math.log(P_MAX)
DELTA_H = (H_MAX - H_MIN) / (N_H - 1)
DELTA_LOGP = (LOGP_MAX - LOGP_MIN) / (N_P - 1)

NC, NS, L = 2, 16, 16
NW = NC * NS
C = 2048
NBLK = C // 128
GROUPS = C // L

LN2 = math.log(2.0)
SQRT2 = math.sqrt(2.0)


def _iota16():
    return lax.broadcasted_iota(jnp.int32, (L,), 0)


def _full16(v, dtype=jnp.int32):
    return jnp.full((L,), v, dtype=dtype)


_LOG_COEFFS = (
    3.342326883898283e-08, 1.0000030986470891, -0.5000129330593639,
    0.3330481239502683, -0.2491121064546301, 0.2061178523961455,
    -0.18627697325343368, 0.11448435452422787,
)


def _log16(b):
    e = (b >> 23) - 127
    m = 1.0 + (b & 0x7FFFFF).astype(jnp.float32) * jnp.float32(2.0 ** -23)
    big = m >= jnp.float32(SQRT2)
    t = jnp.where(big, m * jnp.float32(0.5) - 1.0, m - 1.0)
    e = (e + jnp.where(big, 1, 0)).astype(jnp.float32)
    poly = jnp.float32(_LOG_COEFFS[7])
    for k in range(6, -1, -1):
        poly = poly * t + jnp.float32(_LOG_COEFFS[k])
    return e * jnp.float32(LN2) + poly


def _make_kernel(nq):
    assert nq % (NW * C) == 0
    q_per_w = nq // NW
    g_chunks = q_per_w // C
    assert g_chunks % 2 == 0 and g_chunks >= 2

    mesh = plsc.VectorSubcoreMesh(core_axis_name="c", subcore_axis_name="s")

    @functools.partial(
        pl.kernel,
        out_type=jax.ShapeDtypeStruct((nq,), jnp.float32),
        mesh=mesh,
        scratch_types=[
            pltpu.VMEM((C,), jnp.float32),
            pltpu.VMEM((C,), jnp.float32),
            pltpu.VMEM((C,), jnp.int32),
            pltpu.VMEM((C,), jnp.int32),
            pltpu.VMEM((C,), jnp.float32),
            pltpu.VMEM((C,), jnp.float32),
            pltpu.VMEM((C,), jnp.float32),
            pltpu.VMEM((C,), jnp.float32),
            pltpu.VMEM((NBLK, 128), jnp.int32),
            pltpu.VMEM((NBLK, 128), jnp.int32),
            pltpu.VMEM((NBLK, 128, L), jnp.float32),
            pltpu.VMEM((NBLK, 128, L), jnp.float32),
            pltpu.VMEM((C,), jnp.float32),
            pltpu.VMEM((C,), jnp.float32),
            pltpu.SemaphoreType.DMA,
            pltpu.SemaphoreType.DMA,
            pltpu.SemaphoreType.DMA,
            pltpu.SemaphoreType.DMA,
            pltpu.SemaphoreType.DMA,
            pltpu.SemaphoreType.DMA,
        ],
        compiler_params=pltpu.CompilerParams(
            needs_layout_passes=False, use_tc_tiling_on_sc=False),
    )
    def bicubic(h_hbm, p_hbm, tab_hbm, out_hbm,
                h0, h1, p0, p1, x0, x1, y0, y1, i0, i1, r0b, r1b, o0, o1,
                lsem0, lsem1, gsem0, gsem1, osem0, osem1):
        wid = lax.axis_index("s") * NC + lax.axis_index("c")
        base = wid * q_per_w

        sets = [
            dict(h=h0, p=p0, x=x0, y=y0, idx=i0, rows=r0b, o=o0,
                 lsem=lsem0, gsem=gsem0, osem=osem0),
            dict(h=h1, p=p1, x=x1, y=y1, idx=i1, rows=r1b, o=o1,
                 lsem=lsem1, gsem=gsem1, osem=osem1),
        ]

        inv_dh = jnp.float32(1.0 / DELTA_H)
        inv_dlp = jnp.float32(1.0 / DELTA_LOGP)
        hmin = jnp.float32(H_MIN)
        lpmin = jnp.float32(LOGP_MIN)

        def load(g, s):
            start = base + g * C
            pltpu.async_copy(h_hbm.at[pl.ds(start, C)], s["h"], s["lsem"])
            pltpu.async_copy(p_hbm.at[pl.ds(start, C)], s["p"], s["lsem"])

        def stage(g, s):
            start = base + g * C
            pltpu.make_async_copy(
                h_hbm.at[pl.ds(start, C)], s["h"], s["lsem"]).wait()
            pltpu.make_async_copy(
                p_hbm.at[pl.ds(start, C)], s["p"], s["lsem"]).wait()

            @plsc.parallel_loop(0, GROUPS, 1, unroll=4)
            def _(t):
                o = t * L
                h16 = s["h"][pl.ds(o, L)]
                p16 = s["p"][pl.ds(o, L)]
                xi = (h16 - hmin) * inv_dh
                ii = jnp.minimum(jnp.maximum(xi.astype(jnp.int32), 0), N_H - 2)
                x = xi - ii.astype(jnp.float32)
                yj = (_log16(p16) - lpmin) * inv_dlp
                jv = jnp.minimum(jnp.maximum(yj.astype(jnp.int32), 0), N_P - 2)
                y = yj - jv.astype(jnp.float32)
                s["x"][pl.ds(o, L)] = x
                s["y"][pl.ds(o, L)] = y
                s["idx"][t >> 3, pl.ds((t & 7) * L, L)] = (
                    ((ii >> 3) << 13) + (jv << 3) + (ii & 7))

            for blk in range(NBLK):
                pltpu.async_copy(
                    tab_hbm.at[s["idx"].at[blk]], s["rows"].at[blk], s["gsem"])

        def finish(g, s, drain):
            if drain:
                pltpu.make_async_copy(
                    s["o"], out_hbm.at[pl.ds(base, C)], s["osem"]).wait()
            for blk in range(NBLK):
                pltpu.make_async_copy(
                    tab_hbm.at[s["idx"].at[blk]], s["rows"].at[blk], s["gsem"]
                ).wait()

            @plsc.parallel_loop(0, GROUPS, 1, unroll=4)
            def _(t):
                o = t * L
                x = s["x"][pl.ds(o, L)]
                y = s["y"][pl.ds(o, L)]
                blk = _full16(0) + (t >> 3)
                qid = _iota16() + (t & 7) * L
                c = [
                    plsc.load_gather(s["rows"], [blk, qid, _full16(k)])
                    for k in range(16)
                ]
                r0 = c[0] + x * (c[1] + x * (c[2] + x * c[3]))
                r1 = c[4] + x * (c[5] + x * (c[6] + x * c[7]))
                r2 = c[8] + x * (c[9] + x * (c[10] + x * c[11]))
                r3 = c[12] + x * (c[13] + x * (c[14] + x * c[15]))
                s["o"][pl.ds(o, L)] = r0 + y * (r1 + y * (r2 + y * r3))
            pltpu.async_copy(s["o"], out_hbm.at[pl.ds(base + g * C, C)],
                             s["osem"])

        def fused(g, s):
            pltpu.make_async_copy(
                s["o"], out_hbm.at[pl.ds(base, C)], s["osem"]).wait()
            for blk in range(NBLK):
                pltpu.make_async_copy(
                    tab_hbm.at[s["idx"].at[blk]], s["rows"].at[blk], s["gsem"]
                ).wait()
            start = base + (g + 2) * C
            pltpu.make_async_copy(
                h_hbm.at[pl.ds(start, C)], s["h"], s["lsem"]).wait()
            pltpu.make_async_copy(
                p_hbm.at[pl.ds(start, C)], s["p"], s["lsem"]).wait()

            @plsc.parallel_loop(0, GROUPS, 1, unroll=4)
            def _(t):
                o = t * L
                x = s["x"][pl.ds(o, L)]
                y = s["y"][pl.ds(o, L)]
                blk = _full16(0) + (t >> 3)
                qid = _iota16() + (t & 7) * L
                c = [
                    plsc.load_gather(s["rows"], [blk, qid, _full16(k)])
                    for k in range(16)
                ]
                r0 = c[0] + x * (c[1] + x * (c[2] + x * c[3]))
                r1 = c[4] + x * (c[5] + x * (c[6] + x * c[7]))
                r2 = c[8] + x * (c[9] + x * (c[10] + x * c[11]))
                r3 = c[12] + x * (c[13] + x * (c[14] + x * c[15]))
                s["o"][pl.ds(o, L)] = r0 + y * (r1 + y * (r2 + y * r3))
                h16 = s["h"][pl.ds(o, L)]
                p16 = s["p"][pl.ds(o, L)]
                xi = (h16 - hmin) * inv_dh
                ii = jnp.minimum(jnp.maximum(xi.astype(jnp.int32), 0), N_H - 2)
                s["x"][pl.ds(o, L)] = xi - ii.astype(jnp.float32)
                yj = (_log16(p16) - lpmin) * inv_dlp
                jv = jnp.minimum(jnp.maximum(yj.astype(jnp.int32), 0), N_P - 2)
                s["y"][pl.ds(o, L)] = yj - jv.astype(jnp.float32)
                s["idx"][t >> 3, pl.ds((t & 7) * L, L)] = (
                    ((ii >> 3) << 13) + (jv << 3) + (ii & 7))

            pltpu.async_copy(s["o"], out_hbm.at[pl.ds(base + g * C, C)],
                             s["osem"])
            for blk in range(NBLK):
                pltpu.async_copy(
                    tab_hbm.at[s["idx"].at[blk]], s["rows"].at[blk], s["gsem"])

        G = g_chunks
        assert G >= 8

        load(0, sets[0])
        load(1, sets[1])
        stage(0, sets[0])
        load(2, sets[0])
        stage(1, sets[1])
        load(3, sets[1])
        finish(0, sets[0], drain=False)
        stage(2, sets[0])
        load(4, sets[0])
        finish(1, sets[1], drain=False)
        stage(3, sets[1])
        load(5, sets[1])

        def outer(gg, _):
            g = gg * 2
            fused(g, sets[0])
            load(g + 4, sets[0])
            fused(g + 1, sets[1])
            load(g + 5, sets[1])
            return 0

        lax.fori_loop(1, (G - 4) // 2, outer, 0)

        finish(G - 4, sets[0], drain=True)
        stage(G - 2, sets[0])
        finish(G - 3, sets[1], drain=True)
        stage(G - 1, sets[1])
        finish(G - 2, sets[0], drain=True)
        finish(G - 1, sets[1], drain=True)
        pltpu.make_async_copy(
            sets[0]["o"], out_hbm.at[pl.ds(base, C)], sets[0]["osem"]).wait()
        pltpu.make_async_copy(
            sets[1]["o"], out_hbm.at[pl.ds(base, C)], sets[1]["osem"]).wait()

    return bicubic


def _detile(ct):
    n_blocks = (ct.shape[0] + 7) // 8

    def body(x_ref, o_ref):
        x = x_ref[...]
        o_ref[...] = x.reshape(128, N_P).T

    return pl.pallas_call(
        body,
        grid=(n_blocks,),
        in_specs=[pl.BlockSpec((8, 16, N_P), lambda g: (g, 0, 0))],
        out_specs=pl.BlockSpec((N_P, 128), lambda g: (g, 0)),
        out_shape=jax.ShapeDtypeStruct((n_blocks * N_P, 128), jnp.float32),
        compiler_params=pltpu.CompilerParams(
            dimension_semantics=("parallel",)),
    )(ct)


@jax.jit
def kernel(h, p, coeffs):
    nq = h.shape[0]
    ct = jnp.transpose(coeffs, (0, 2, 1))
    table = _detile(ct).reshape(-1, 16)
    pbits = lax.bitcast_convert_type(p, jnp.int32)
    return _make_kernel(nq)(h, pbits, table)


# speedup vs baseline: 1.0864x; 1.0864x over previous
import functools
import math

import jax
import jax.numpy as jnp
from jax import lax
from jax.experimental import pallas as pl
from jax.experimental.pallas import tpu as pltpu
from jax.experimental.pallas import tpu_sc as plsc

H_MIN, H_MAX = 2.0e5, 3.5e6
P_MIN, P_MAX = 1.0e3, 1.0e8
N_H, N_P = 1024, 1024
LOGP_MIN = math.log(P_MIN)
LOGP_MAX = math.log(P_MAX)
DELTA_H = (H_MAX - H_MIN) / (N_H - 1)
DELTA_LOGP = (LOGP_MAX - LOGP_MIN) / (N_P - 1)

NC, NS, L = 2, 16, 16
NW = NC * NS
C = 2048
NBLK = C // 128
GROUPS = C // L

LN2 = math.log(2.0)
SQRT2 = math.sqrt(2.0)


def _iota16():
    return lax.broadcasted_iota(jnp.int32, (L,), 0)


def _full16(v, dtype=jnp.int32):
    return jnp.full((L,), v, dtype=dtype)


_LOG_COEFFS = (
    3.342326883898283e-08, 1.0000030986470891, -0.5000129330593639,
    0.3330481239502683, -0.2491121064546301, 0.2061178523961455,
    -0.18627697325343368, 0.11448435452422787,
)


def _log16(b):
    e = (b >> 23) - 127
    m = 1.0 + (b & 0x7FFFFF).astype(jnp.float32) * jnp.float32(2.0 ** -23)
    big = m >= jnp.float32(SQRT2)
    t = jnp.where(big, m * jnp.float32(0.5) - 1.0, m - 1.0)
    e = (e + jnp.where(big, 1, 0)).astype(jnp.float32)
    poly = jnp.float32(_LOG_COEFFS[7])
    for k in range(6, -1, -1):
        poly = poly * t + jnp.float32(_LOG_COEFFS[k])
    return e * jnp.float32(LN2) + poly


def _make_kernel(nq):
    assert nq % (NW * C) == 0
    q_per_w = nq // NW
    g_chunks = q_per_w // C
    assert g_chunks % 2 == 0 and g_chunks >= 2

    mesh = plsc.VectorSubcoreMesh(core_axis_name="c", subcore_axis_name="s")

    @functools.partial(
        pl.kernel,
        out_type=jax.ShapeDtypeStruct((nq,), jnp.float32),
        mesh=mesh,
        scratch_types=[
            pltpu.VMEM((C,), jnp.float32),
            pltpu.VMEM((C,), jnp.float32),
            pltpu.VMEM((C,), jnp.int32),
            pltpu.VMEM((C,), jnp.int32),
            pltpu.VMEM((C,), jnp.float32),
            pltpu.VMEM((C,), jnp.float32),
            pltpu.VMEM((C,), jnp.float32),
            pltpu.VMEM((C,), jnp.float32),
            pltpu.VMEM((NBLK, 128), jnp.int32),
            pltpu.VMEM((NBLK, 128), jnp.int32),
            pltpu.VMEM((NBLK, 128, L), jnp.float32),
            pltpu.VMEM((NBLK, 128, L), jnp.float32),
            pltpu.VMEM((C,), jnp.float32),
            pltpu.VMEM((C,), jnp.float32),
            pltpu.SemaphoreType.DMA,
            pltpu.SemaphoreType.DMA,
            pltpu.SemaphoreType.DMA,
            pltpu.SemaphoreType.DMA,
            pltpu.SemaphoreType.DMA,
            pltpu.SemaphoreType.DMA,
        ],
        compiler_params=pltpu.CompilerParams(
            needs_layout_passes=False, use_tc_tiling_on_sc=False),
    )
    def bicubic(h_hbm, p_hbm, tab_hbm, out_hbm,
                h0, h1, p0, p1, x0, x1, y0, y1, i0, i1, r0b, r1b, o0, o1,
                lsem0, lsem1, gsem0, gsem1, osem0, osem1):
        wid = lax.axis_index("s") * NC + lax.axis_index("c")
        base = wid * q_per_w

        sets = [
            dict(h=h0, p=p0, x=x0, y=y0, idx=i0, rows=r0b, o=o0,
                 lsem=lsem0, gsem=gsem0, osem=osem0),
            dict(h=h1, p=p1, x=x1, y=y1, idx=i1, rows=r1b, o=o1,
                 lsem=lsem1, gsem=gsem1, osem=osem1),
        ]

        inv_dh = jnp.float32(1.0 / DELTA_H)
        inv_dlp = jnp.float32(1.0 / DELTA_LOGP)
        hmin = jnp.float32(H_MIN)
        lpmin = jnp.float32(LOGP_MIN)

        def load(g, s):
            start = base + g * C
            pltpu.async_copy(h_hbm.at[pl.ds(start, C)], s["h"], s["lsem"])
            pltpu.async_copy(p_hbm.at[pl.ds(start, C)], s["p"], s["lsem"])

        def stage(g, s):
            start = base + g * C
            pltpu.make_async_copy(
                h_hbm.at[pl.ds(start, C)], s["h"], s["lsem"]).wait()
            pltpu.make_async_copy(
                p_hbm.at[pl.ds(start, C)], s["p"], s["lsem"]).wait()

            @plsc.parallel_loop(0, GROUPS, 1, unroll=4)
            def _(t):
                o = t * L
                h16 = s["h"][pl.ds(o, L)]
                p16 = s["p"][pl.ds(o, L)]
                xi = (h16 - hmin) * inv_dh
                ii = jnp.minimum(jnp.maximum(xi.astype(jnp.int32), 0), N_H - 2)
                x = xi - ii.astype(jnp.float32)
                yj = (_log16(p16) - lpmin) * inv_dlp
                jv = jnp.minimum(jnp.maximum(yj.astype(jnp.int32), 0), N_P - 2)
                y = yj - jv.astype(jnp.float32)
                s["x"][pl.ds(o, L)] = x
                s["y"][pl.ds(o, L)] = y
                s["idx"][t >> 3, pl.ds((t & 7) * L, L)] = (
                    ((ii >> 3) << 13) + (jv << 3) + (ii & 7))

            for blk in range(NBLK):
                pltpu.async_copy(
                    tab_hbm.at[s["idx"].at[blk]], s["rows"].at[blk], s["gsem"])

        def finish(g, s, drain):
            if drain:
                pltpu.make_async_copy(
                    s["o"], out_hbm.at[pl.ds(base, C)], s["osem"]).wait()
            for blk in range(NBLK):
                pltpu.make_async_copy(
                    tab_hbm.at[s["idx"].at[blk]], s["rows"].at[blk], s["gsem"]
                ).wait()

            @plsc.parallel_loop(0, GROUPS, 1, unroll=4)
            def _(t):
                o = t * L
                x = s["x"][pl.ds(o, L)]
                y = s["y"][pl.ds(o, L)]
                blk = _full16(0) + (t >> 3)
                qid = _iota16() + (t & 7) * L
                c = [
                    plsc.load_gather(s["rows"], [blk, qid, _full16(k)])
                    for k in range(16)
                ]
                r0 = c[0] + x * (c[1] + x * (c[2] + x * c[3]))
                r1 = c[4] + x * (c[5] + x * (c[6] + x * c[7]))
                r2 = c[8] + x * (c[9] + x * (c[10] + x * c[11]))
                r3 = c[12] + x * (c[13] + x * (c[14] + x * c[15]))
                s["o"][pl.ds(o, L)] = r0 + y * (r1 + y * (r2 + y * r3))
            pltpu.async_copy(s["o"], out_hbm.at[pl.ds(base + g * C, C)],
                             s["osem"])

        def fused(g, s):
            pltpu.make_async_copy(
                s["o"], out_hbm.at[pl.ds(base, C)], s["osem"]).wait()
            for blk in range(NBLK):
                pltpu.make_async_copy(
                    tab_hbm.at[s["idx"].at[blk]], s["rows"].at[blk], s["gsem"]
                ).wait()
            start = base + (g + 2) * C
            pltpu.make_async_copy(
                h_hbm.at[pl.ds(start, C)], s["h"], s["lsem"]).wait()
            pltpu.make_async_copy(
                p_hbm.at[pl.ds(start, C)], s["p"], s["lsem"]).wait()

            @plsc.parallel_loop(0, GROUPS, 1, unroll=2)
            def _(t):
                o = t * L
                x = s["x"][pl.ds(o, L)]
                y = s["y"][pl.ds(o, L)]
                blk = _full16(0) + (t >> 3)
                qid = _iota16() + (t & 7) * L
                c = [
                    plsc.load_gather(s["rows"], [blk, qid, _full16(k)])
                    for k in range(16)
                ]
                r0 = c[0] + x * (c[1] + x * (c[2] + x * c[3]))
                r1 = c[4] + x * (c[5] + x * (c[6] + x * c[7]))
                r2 = c[8] + x * (c[9] + x * (c[10] + x * c[11]))
                r3 = c[12] + x * (c[13] + x * (c[14] + x * c[15]))
                s["o"][pl.ds(o, L)] = r0 + y * (r1 + y * (r2 + y * r3))
                h16 = s["h"][pl.ds(o, L)]
                p16 = s["p"][pl.ds(o, L)]
                xi = (h16 - hmin) * inv_dh
                ii = jnp.minimum(jnp.maximum(xi.astype(jnp.int32), 0), N_H - 2)
                s["x"][pl.ds(o, L)] = xi - ii.astype(jnp.float32)
                yj = (_log16(p16) - lpmin) * inv_dlp
                jv = jnp.minimum(jnp.maximum(yj.astype(jnp.int32), 0), N_P - 2)
                s["y"][pl.ds(o, L)] = yj - jv.astype(jnp.float32)
                s["idx"][t >> 3, pl.ds((t & 7) * L, L)] = (
                    ((ii >> 3) << 13) + (jv << 3) + (ii & 7))

            pltpu.async_copy(s["o"], out_hbm.at[pl.ds(base + g * C, C)],
                             s["osem"])
            for blk in range(NBLK):
                pltpu.async_copy(
                    tab_hbm.at[s["idx"].at[blk]], s["rows"].at[blk], s["gsem"])

        G = g_chunks
        assert G >= 8

        load(0, sets[0])
        load(1, sets[1])
        stage(0, sets[0])
        load(2, sets[0])
        stage(1, sets[1])
        load(3, sets[1])
        finish(0, sets[0], drain=False)
        stage(2, sets[0])
        load(4, sets[0])
        finish(1, sets[1], drain=False)
        stage(3, sets[1])
        load(5, sets[1])

        def outer(gg, _):
            g = gg * 2
            fused(g, sets[0])
            load(g + 4, sets[0])
            fused(g + 1, sets[1])
            load(g + 5, sets[1])
            return 0

        lax.fori_loop(1, (G - 4) // 2, outer, 0)

        finish(G - 4, sets[0], drain=True)
        stage(G - 2, sets[0])
        finish(G - 3, sets[1], drain=True)
        stage(G - 1, sets[1])
        finish(G - 2, sets[0], drain=True)
        finish(G - 1, sets[1], drain=True)
        pltpu.make_async_copy(
            sets[0]["o"], out_hbm.at[pl.ds(base, C)], sets[0]["osem"]).wait()
        pltpu.make_async_copy(
            sets[1]["o"], out_hbm.at[pl.ds(base, C)], sets[1]["osem"]).wait()

    return bicubic


def _detile(ct):
    n_blocks = (ct.shape[0] + 7) // 8

    def body(x_ref, o_ref):
        x = x_ref[...]
        o_ref[...] = x.reshape(128, N_P).T

    return pl.pallas_call(
        body,
        grid=(n_blocks,),
        in_specs=[pl.BlockSpec((8, 16, N_P), lambda g: (g, 0, 0))],
        out_specs=pl.BlockSpec((N_P, 128), lambda g: (g, 0)),
        out_shape=jax.ShapeDtypeStruct((n_blocks * N_P, 128), jnp.float32),
        compiler_params=pltpu.CompilerParams(
            dimension_semantics=("parallel",)),
    )(ct)


@jax.jit
def kernel(h, p, coeffs):
    nq = h.shape[0]
    ct = jnp.transpose(coeffs, (0, 2, 1))
    table = _detile(ct).reshape(-1, 16)
    pbits = lax.bitcast_convert_type(p, jnp.int32)
    return _make_kernel(nq)(h, pbits, table)
